# baseline (speedup 1.0000x reference)
import functools

import jax
import jax.numpy as jnp
from jax import lax
from jax.experimental import pallas as pl

NUM_PTS = 2048
HF = 128
WF = 128
HC = 32
WC = 32
NC = 19
CF = 192
NPIX = HF * WF
BLK = 8192
NBLK = NPIX // BLK


def _upsample_matrices():
    def mat(out_n, in_n):
        s = jnp.linspace(0.0, in_n - 1.0, out_n)
        i0 = jnp.clip(jnp.floor(s), 0, in_n - 1)
        i1 = jnp.clip(i0 + 1, 0, in_n - 1)
        w = (s - i0)[:, None]
        oh0 = jax.nn.one_hot(i0.astype(jnp.int32), in_n, dtype=jnp.float32)
        oh1 = jax.nn.one_hot(i1.astype(jnp.int32), in_n, dtype=jnp.float32)
        return oh0 * (1.0 - w) + oh1 * w

    wy = mat(HF, HC)
    wxt = mat(WF, WC).T
    return wy, wxt


def _select_kernel(coarse_ref, wy_ref, wxt_ref, out_ref):
    wy = wy_ref[...]
    wxt = wxt_ref[...]
    ups = []
    for ci in range(NC):
        a1 = jnp.dot(coarse_ref[0, ci], wxt, precision=lax.Precision.HIGHEST,
                     preferred_element_type=jnp.float32)
        m = jnp.dot(wy, a1, precision=lax.Precision.HIGHEST,
                    preferred_element_type=jnp.float32)
        ups.append(m)
        out_ref[0, ci] = m
    cmax = functools.reduce(jnp.maximum, ups)
    s = functools.reduce(jnp.add, [jnp.exp(u - cmax) for u in ups])
    unc = -(1.0 / s)
    keys = -lax.bitcast_convert_type(unc, jnp.int32)

    kpts = jnp.int32(NUM_PTS)
    kflat = keys.reshape(1, NPIX)
    io16 = lax.broadcasted_iota(jnp.int32, (16, 1), 0) + 1

    lo = jnp.min(keys)
    hi = jnp.max(keys) + 1

    def body_val(_, carry):
        lo, hi = carry
        step = (hi - lo + 15) // 16
        ts = lo + step * io16
        cnt = jnp.sum((kflat >= ts).astype(jnp.int32), axis=1)
        num_ok = jnp.sum((cnt >= kpts).astype(jnp.int32))
        return (lo + num_ok * step,
                jnp.minimum(hi, lo + (num_ok + 1) * step))

    lo, hi = lax.fori_loop(0, 7, body_val, (lo, hi))
    kth = lo
    n_gt = jnp.sum((keys >= kth + 1).astype(jnp.int32))
    need = kpts - n_gt

    iy = lax.broadcasted_iota(jnp.int32, (HF, WF), 0)
    ix = lax.broadcasted_iota(jnp.int32, (HF, WF), 1)
    idx = iy * WF + ix
    eq = keys == kth
    eqidx = jnp.where(eq, idx, jnp.int32(NPIX)).reshape(1, NPIX)

    def body_idx(_, carry):
        lo2, hi2 = carry
        step = (hi2 - lo2 + 15) // 16
        ts = lo2 + step * io16
        cnt = jnp.sum((eqidx <= ts).astype(jnp.int32), axis=1)
        num_lt = jnp.sum((cnt < need).astype(jnp.int32))
        return (lo2 + num_lt * step,
                jnp.minimum(hi2, lo2 + (num_lt + 1) * step))

    lo2, hi2 = lax.fori_loop(0, 4, body_idx,
                             (jnp.int32(-1), jnp.int32(NPIX - 1)))
    jthr = hi2
    mask = (keys > kth) | (eq & (idx <= jthr))
    out_ref[0, NC] = mask.astype(jnp.float32)


def _mlp_kernel(fine_ref, aux_ref, w1f_ref, w1c_ref, w2_ref, w3_ref, b1_ref,
                b2_ref, b3_ref, out_ref):
    fine = fine_ref[0].reshape(CF, BLK)
    aux = aux_ref[0].reshape(NC + 1, BLK)
    cu = aux[:NC]
    msk = aux[NC:NC + 1]
    h = (jnp.dot(w1f_ref[...], fine.astype(jnp.bfloat16),
                 preferred_element_type=jnp.float32)
         + jnp.dot(w1c_ref[...], cu.astype(jnp.bfloat16),
                   preferred_element_type=jnp.float32))
    h = jnp.maximum(h + b1_ref[...], 0.0)
    h = jnp.dot(w2_ref[...], h.astype(jnp.bfloat16),
                preferred_element_type=jnp.float32)
    h = jnp.maximum(h + b2_ref[...], 0.0)
    y = jnp.dot(w3_ref[...], h.astype(jnp.bfloat16),
                preferred_element_type=jnp.float32)
    y = y + b3_ref[...]
    out_ref[0] = jnp.where(msk > 0.0, y, cu).reshape(NC, BLK // WF, WF)


def kernel(coarse_logits, fine_features, W1, b1, W2, b2, W3, b3):
    B = coarse_logits.shape[0]
    wy, wxt = _upsample_matrices()

    aux4 = pl.pallas_call(
        _select_kernel,
        grid=(B,),
        in_specs=[
            pl.BlockSpec((1, NC, HC, WC), lambda b: (b, 0, 0, 0)),
            pl.BlockSpec((HF, HC), lambda b: (0, 0)),
            pl.BlockSpec((HC, WF), lambda b: (0, 0)),
        ],
        out_specs=pl.BlockSpec((1, NC + 1, HF, WF), lambda b: (b, 0, 0, 0)),
        out_shape=jax.ShapeDtypeStruct((B, NC + 1, HF, WF), jnp.float32),
    )(coarse_logits, wy, wxt)

    w1f = W1[:, :CF].astype(jnp.bfloat16)
    w1c = W1[:, CF:].astype(jnp.bfloat16)
    w2b = W2.astype(jnp.bfloat16)
    w3b = W3.astype(jnp.bfloat16)
    b1r = b1[:, None]
    b2r = b2[:, None]
    b3r = b3[:, None]

    out = pl.pallas_call(
        _mlp_kernel,
        grid=(B, NBLK),
        in_specs=[
            pl.BlockSpec((1, CF, BLK // WF, WF), lambda b, j: (b, 0, j, 0)),
            pl.BlockSpec((1, NC + 1, BLK // WF, WF), lambda b, j: (b, 0, j, 0)),
            pl.BlockSpec((256, CF), lambda b, j: (0, 0)),
            pl.BlockSpec((256, NC), lambda b, j: (0, 0)),
            pl.BlockSpec((256, 256), lambda b, j: (0, 0)),
            pl.BlockSpec((NC, 256), lambda b, j: (0, 0)),
            pl.BlockSpec((256, 1), lambda b, j: (0, 0)),
            pl.BlockSpec((256, 1), lambda b, j: (0, 0)),
            pl.BlockSpec((NC, 1), lambda b, j: (0, 0)),
        ],
        out_specs=pl.BlockSpec((1, NC, BLK // WF, WF), lambda b, j: (b, 0, j, 0)),
        out_shape=jax.ShapeDtypeStruct((B, NC, HF, WF), jnp.float32),
    )(fine_features, aux4, w1f, w1c, w2b, w3b, b1r, b2r, b3r)

    return out


# speedup vs baseline: 1.0528x; 1.0528x over previous
import functools

import jax
import jax.numpy as jnp
from jax import lax
from jax.experimental import pallas as pl

NUM_PTS = 2048
HF = 128
WF = 128
HC = 32
WC = 32
NC = 19
CF = 192
NPIX = HF * WF
BLK = 8192
NBLK = NPIX // BLK


def _upsample_matrices():
    def mat(out_n, in_n):
        s = jnp.linspace(0.0, in_n - 1.0, out_n)
        i0 = jnp.clip(jnp.floor(s), 0, in_n - 1)
        i1 = jnp.clip(i0 + 1, 0, in_n - 1)
        w = (s - i0)[:, None]
        oh0 = jax.nn.one_hot(i0.astype(jnp.int32), in_n, dtype=jnp.float32)
        oh1 = jax.nn.one_hot(i1.astype(jnp.int32), in_n, dtype=jnp.float32)
        return oh0 * (1.0 - w) + oh1 * w

    wy = mat(HF, HC)
    wxt = mat(WF, WC).T
    return wy, wxt


def _select_kernel(coarse_ref, wy_ref, wxt_ref, out_ref):
    wy = wy_ref[...]
    wxt = wxt_ref[...]
    ups = []
    for ci in range(NC):
        a1 = jnp.dot(coarse_ref[0, ci], wxt, precision=lax.Precision.HIGHEST,
                     preferred_element_type=jnp.float32)
        m = jnp.dot(wy, a1, precision=lax.Precision.HIGHEST,
                    preferred_element_type=jnp.float32)
        ups.append(m)
        out_ref[0, ci] = m
    cmax = functools.reduce(jnp.maximum, ups)
    s = functools.reduce(jnp.add, [jnp.exp(u - cmax) for u in ups])
    unc = -(1.0 / s)
    keys = -lax.bitcast_convert_type(unc, jnp.int32)

    kpts = jnp.int32(NUM_PTS)
    kflat = keys.reshape(1, NPIX)
    io16 = lax.broadcasted_iota(jnp.int32, (16, 1), 0) + 1

    lo = jnp.min(keys)
    hi = jnp.max(keys) + 1

    def body_val(_, carry):
        lo, hi = carry
        step = (hi - lo + 15) // 16
        ts = lo + step * io16
        cnt = jnp.sum((kflat >= ts).astype(jnp.int32), axis=1)
        num_ok = jnp.sum((cnt >= kpts).astype(jnp.int32))
        return (lo + num_ok * step,
                jnp.minimum(hi, lo + (num_ok + 1) * step))

    lo, hi = lax.fori_loop(0, 7, body_val, (lo, hi))
    kth = lo
    n_gt = jnp.sum((keys >= kth + 1).astype(jnp.int32))
    need = kpts - n_gt

    iy = lax.broadcasted_iota(jnp.int32, (HF, WF), 0)
    ix = lax.broadcasted_iota(jnp.int32, (HF, WF), 1)
    idx = iy * WF + ix
    eq = keys == kth
    eqidx = jnp.where(eq, idx, jnp.int32(NPIX)).reshape(1, NPIX)

    def body_idx(_, carry):
        lo2, hi2 = carry
        step = (hi2 - lo2 + 15) // 16
        ts = lo2 + step * io16
        cnt = jnp.sum((eqidx <= ts).astype(jnp.int32), axis=1)
        num_lt = jnp.sum((cnt < need).astype(jnp.int32))
        return (lo2 + num_lt * step,
                jnp.minimum(hi2, lo2 + (num_lt + 1) * step))

    lo2, hi2 = lax.fori_loop(0, 4, body_idx,
                             (jnp.int32(-1), jnp.int32(NPIX - 1)))
    jthr = hi2
    mask = (keys > kth) | (eq & (idx <= jthr))
    out_ref[0, NC] = mask.astype(jnp.float32)


def _mlp_kernel(fine_ref, aux_ref, w1_ref, w2_ref, w3_ref, b1_ref, b2_ref,
                b3_ref, out_ref):
    fine = fine_ref[0].reshape(CF, BLK)
    aux = aux_ref[0].reshape(NC + 1, BLK)
    cu = aux[:NC]
    msk = aux[NC:NC + 1]
    h = (jnp.dot(w1_ref[:, :CF].astype(jnp.bfloat16),
                 fine.astype(jnp.bfloat16),
                 preferred_element_type=jnp.float32)
         + jnp.dot(w1_ref[:, CF:].astype(jnp.bfloat16),
                   cu.astype(jnp.bfloat16),
                   preferred_element_type=jnp.float32))
    h = jnp.maximum(h + b1_ref[...], 0.0)
    h = jnp.dot(w2_ref[...].astype(jnp.bfloat16), h.astype(jnp.bfloat16),
                preferred_element_type=jnp.float32)
    h = jnp.maximum(h + b2_ref[...], 0.0)
    y = jnp.dot(w3_ref[...].astype(jnp.bfloat16), h.astype(jnp.bfloat16),
                preferred_element_type=jnp.float32)
    y = y + b3_ref[...]
    out_ref[0] = jnp.where(msk > 0.0, y, cu).reshape(NC, BLK // WF, WF)


def kernel(coarse_logits, fine_features, W1, b1, W2, b2, W3, b3):
    B = coarse_logits.shape[0]
    wy, wxt = _upsample_matrices()

    aux4 = pl.pallas_call(
        _select_kernel,
        grid=(B,),
        in_specs=[
            pl.BlockSpec((1, NC, HC, WC), lambda b: (b, 0, 0, 0)),
            pl.BlockSpec((HF, HC), lambda b: (0, 0)),
            pl.BlockSpec((HC, WF), lambda b: (0, 0)),
        ],
        out_specs=pl.BlockSpec((1, NC + 1, HF, WF), lambda b: (b, 0, 0, 0)),
        out_shape=jax.ShapeDtypeStruct((B, NC + 1, HF, WF), jnp.float32),
    )(coarse_logits, wy, wxt)


    b1r = b1[:, None]
    b2r = b2[:, None]
    b3r = b3[:, None]

    out = pl.pallas_call(
        _mlp_kernel,
        grid=(B, NBLK),
        in_specs=[
            pl.BlockSpec((1, CF, BLK // WF, WF), lambda b, j: (b, 0, j, 0)),
            pl.BlockSpec((1, NC + 1, BLK // WF, WF), lambda b, j: (b, 0, j, 0)),
            pl.BlockSpec((256, 211), lambda b, j: (0, 0)),
            pl.BlockSpec((256, 256), lambda b, j: (0, 0)),
            pl.BlockSpec((NC, 256), lambda b, j: (0, 0)),
            pl.BlockSpec((256, 1), lambda b, j: (0, 0)),
            pl.BlockSpec((256, 1), lambda b, j: (0, 0)),
            pl.BlockSpec((NC, 1), lambda b, j: (0, 0)),
        ],
        out_specs=pl.BlockSpec((1, NC, BLK // WF, WF), lambda b, j: (b, 0, j, 0)),
        out_shape=jax.ShapeDtypeStruct((B, NC, HF, WF), jnp.float32),
    )(fine_features, aux4, W1, W2, W3, b1r, b2r, b3r)

    return out


# speedup vs baseline: 1.1589x; 1.1008x over previous
import functools

import jax
import jax.numpy as jnp
from jax import lax
from jax.experimental import pallas as pl

NUM_PTS = 2048
HF = 128
WF = 128
HC = 32
WC = 32
NC = 19
CF = 192
NPIX = HF * WF
BLK = 8192
NBLK = NPIX // BLK


def _upsample_matrices():
    def mat(out_n, in_n):
        s = jnp.linspace(0.0, in_n - 1.0, out_n)
        i0 = jnp.clip(jnp.floor(s), 0, in_n - 1)
        i1 = jnp.clip(i0 + 1, 0, in_n - 1)
        w = (s - i0)[:, None]
        oh0 = jax.nn.one_hot(i0.astype(jnp.int32), in_n, dtype=jnp.float32)
        oh1 = jax.nn.one_hot(i1.astype(jnp.int32), in_n, dtype=jnp.float32)
        return oh0 * (1.0 - w) + oh1 * w

    wy = mat(HF, HC)
    wxt = mat(WF, WC).T
    return wy, wxt


def _select_kernel(coarse_ref, wy_ref, wxt_ref, out_ref):
    wy = wy_ref[...]
    wxt = wxt_ref[...]
    ups = []
    for ci in range(NC):
        a1 = jnp.dot(coarse_ref[0, ci], wxt, precision=lax.Precision.HIGHEST,
                     preferred_element_type=jnp.float32)
        m = jnp.dot(wy, a1, precision=lax.Precision.HIGHEST,
                    preferred_element_type=jnp.float32)
        ups.append(m)
        out_ref[0, ci] = m
    cmax = functools.reduce(jnp.maximum, ups)
    s = functools.reduce(jnp.add, [jnp.exp(u - cmax) for u in ups])
    unc = -(1.0 / s)
    keys = -lax.bitcast_convert_type(unc, jnp.int32)

    kpts = jnp.int32(NUM_PTS)
    kflat = keys.reshape(1, NPIX)
    io16 = lax.broadcasted_iota(jnp.int32, (16, 1), 0) + 1

    lo = jnp.min(keys)
    hi = jnp.max(keys) + 1

    def body_val(_, carry):
        lo, hi = carry
        step = (hi - lo + 15) // 16
        ts = lo + step * io16
        cnt = jnp.sum((kflat >= ts).astype(jnp.int32), axis=1)
        num_ok = jnp.sum((cnt >= kpts).astype(jnp.int32))
        return (lo + num_ok * step,
                jnp.minimum(hi, lo + (num_ok + 1) * step))

    lo, hi = lax.fori_loop(0, 7, body_val, (lo, hi))
    kth = lo
    n_gt = jnp.sum((keys >= kth + 1).astype(jnp.int32))
    need = kpts - n_gt

    iy = lax.broadcasted_iota(jnp.int32, (HF, WF), 0)
    ix = lax.broadcasted_iota(jnp.int32, (HF, WF), 1)
    idx = iy * WF + ix
    eq = keys == kth
    eqidx = jnp.where(eq, idx, jnp.int32(NPIX)).reshape(1, NPIX)

    def body_idx(_, carry):
        lo2, hi2 = carry
        step = (hi2 - lo2 + 15) // 16
        ts = lo2 + step * io16
        cnt = jnp.sum((eqidx <= ts).astype(jnp.int32), axis=1)
        num_lt = jnp.sum((cnt < need).astype(jnp.int32))
        return (lo2 + num_lt * step,
                jnp.minimum(hi2, lo2 + (num_lt + 1) * step))

    lo2, hi2 = lax.fori_loop(0, 4, body_idx,
                             (jnp.int32(-1), jnp.int32(NPIX - 1)))
    jthr = hi2
    mask = (keys > kth) | (eq & (idx <= jthr))
    out_ref[0, NC] = mask.astype(jnp.float32)


def _mlp_kernel(fine_ref, aux_ref, w1_ref, w2_ref, w3_ref, b1_ref, b2_ref,
                b3_ref, out_ref):
    fine = fine_ref[0].reshape(CF, BLK)
    aux = aux_ref[0].reshape(NC + 1, BLK)
    cu = aux[:NC]
    msk = aux[NC:NC + 1]
    pad = jnp.zeros((256 - CF - NC, BLK), jnp.bfloat16)
    x = jnp.concatenate([fine.astype(jnp.bfloat16), cu.astype(jnp.bfloat16),
                         pad], axis=0)
    h = jnp.dot(w1_ref[...].astype(jnp.bfloat16), x,
                preferred_element_type=jnp.float32)
    h = jnp.maximum(h + b1_ref[...], 0.0)
    h = jnp.dot(w2_ref[...].astype(jnp.bfloat16), h.astype(jnp.bfloat16),
                preferred_element_type=jnp.float32)
    h = jnp.maximum(h + b2_ref[...], 0.0)
    y = jnp.dot(w3_ref[...].astype(jnp.bfloat16), h.astype(jnp.bfloat16),
                preferred_element_type=jnp.float32)
    y = y + b3_ref[...]
    out_ref[0] = jnp.where(msk > 0.0, y, cu).reshape(NC, BLK // WF, WF)


def kernel(coarse_logits, fine_features, W1, b1, W2, b2, W3, b3):
    B = coarse_logits.shape[0]
    wy, wxt = _upsample_matrices()

    aux4 = pl.pallas_call(
        _select_kernel,
        grid=(B,),
        in_specs=[
            pl.BlockSpec((1, NC, HC, WC), lambda b: (b, 0, 0, 0)),
            pl.BlockSpec((HF, HC), lambda b: (0, 0)),
            pl.BlockSpec((HC, WF), lambda b: (0, 0)),
        ],
        out_specs=pl.BlockSpec((1, NC + 1, HF, WF), lambda b: (b, 0, 0, 0)),
        out_shape=jax.ShapeDtypeStruct((B, NC + 1, HF, WF), jnp.float32),
    )(coarse_logits, wy, wxt)

    w1p = jnp.pad(W1, ((0, 0), (0, 256 - W1.shape[1])))
    b1r = b1[:, None]
    b2r = b2[:, None]
    b3r = b3[:, None]

    out = pl.pallas_call(
        _mlp_kernel,
        grid=(B, NBLK),
        in_specs=[
            pl.BlockSpec((1, CF, BLK // WF, WF), lambda b, j: (b, 0, j, 0)),
            pl.BlockSpec((1, NC + 1, BLK // WF, WF), lambda b, j: (b, 0, j, 0)),
            pl.BlockSpec((256, 256), lambda b, j: (0, 0)),
            pl.BlockSpec((256, 256), lambda b, j: (0, 0)),
            pl.BlockSpec((NC, 256), lambda b, j: (0, 0)),
            pl.BlockSpec((256, 1), lambda b, j: (0, 0)),
            pl.BlockSpec((256, 1), lambda b, j: (0, 0)),
            pl.BlockSpec((NC, 1), lambda b, j: (0, 0)),
        ],
        out_specs=pl.BlockSpec((1, NC, BLK // WF, WF), lambda b, j: (b, 0, j, 0)),
        out_shape=jax.ShapeDtypeStruct((B, NC, HF, WF), jnp.float32),
    )(fine_features, aux4, w1p, W2, W3, b1r, b2r, b3r)

    return out


# speedup vs baseline: 1.2255x; 1.0575x over previous
import functools

import jax
import jax.numpy as jnp
from jax import lax
from jax.experimental import pallas as pl

NUM_PTS = 2048
HF = 128
WF = 128
HC = 32
WC = 32
NC = 19
CF = 192
NPIX = HF * WF
BLK = 8192
NBLK = NPIX // BLK


def _upsample_matrices():
    def mat(out_n, in_n):
        s = jnp.linspace(0.0, in_n - 1.0, out_n)
        i0 = jnp.clip(jnp.floor(s), 0, in_n - 1)
        i1 = jnp.clip(i0 + 1, 0, in_n - 1)
        w = (s - i0)[:, None]
        oh0 = jax.nn.one_hot(i0.astype(jnp.int32), in_n, dtype=jnp.float32)
        oh1 = jax.nn.one_hot(i1.astype(jnp.int32), in_n, dtype=jnp.float32)
        return oh0 * (1.0 - w) + oh1 * w

    wy = mat(HF, HC)
    wxt = mat(WF, WC).T
    return wy, wxt


def _select_kernel(coarse_ref, wy_ref, wxt_ref, out_ref):
    wy = wy_ref[...]
    wxt = wxt_ref[...]
    kpts = jnp.int32(NUM_PTS)
    io16 = lax.broadcasted_iota(jnp.int32, (16, 1), 0) + 1
    iy = lax.broadcasted_iota(jnp.int32, (HF, WF), 0)
    ix = lax.broadcasted_iota(jnp.int32, (HF, WF), 1)
    idx = iy * WF + ix
    B = coarse_ref.shape[0]

    keys_list, kflat_list = [], []
    for b in range(B):
        ups = []
        for ci in range(NC):
            a1 = jnp.dot(coarse_ref[b, ci], wxt,
                         precision=lax.Precision.HIGHEST,
                         preferred_element_type=jnp.float32)
            m = jnp.dot(wy, a1, precision=lax.Precision.HIGHEST,
                        preferred_element_type=jnp.float32)
            ups.append(m)
            out_ref[b, ci] = m
        cmax = functools.reduce(jnp.maximum, ups)
        s = functools.reduce(jnp.add, [jnp.exp(u - cmax) for u in ups])
        unc = -(1.0 / s)
        keys = -lax.bitcast_convert_type(unc, jnp.int32)
        keys_list.append(keys)
        kflat_list.append(keys.reshape(1, NPIX))

    def rsum(x):
        return jnp.sum(x, axis=(0, 1), keepdims=True)

    los = [jnp.min(kf, axis=(0, 1), keepdims=True) for kf in kflat_list]
    his = [jnp.max(kf, axis=(0, 1), keepdims=True) + 1 for kf in kflat_list]

    def body_val(_, carry):
        los, his = carry
        nlo, nhi = [], []
        for b in range(B):
            lo, hi = los[b], his[b]
            step = (hi - lo + 15) // 16
            ts = lo + step * io16
            cnt = jnp.sum((kflat_list[b] >= ts).astype(jnp.int32),
                          axis=1, keepdims=True)
            num_ok = rsum((cnt >= kpts).astype(jnp.int32))
            nlo.append(lo + num_ok * step)
            nhi.append(jnp.minimum(hi, lo + (num_ok + 1) * step))
        return (tuple(nlo), tuple(nhi))

    los, his = lax.fori_loop(0, 7, body_val, (tuple(los), tuple(his)))

    needs, eqs, eqidxs = [], [], []
    for b in range(B):
        kth = los[b]
        n_gt = rsum((kflat_list[b] >= kth + 1).astype(jnp.int32))
        needs.append(kpts - n_gt)
        eq = keys_list[b] == kth
        eqs.append(eq)
        eqidxs.append(jnp.where(eq, idx, jnp.int32(NPIX)).reshape(1, NPIX))

    def body_idx(_, carry):
        los2, his2 = carry
        nlo, nhi = [], []
        for b in range(B):
            lo2, hi2 = los2[b], his2[b]
            step = (hi2 - lo2 + 15) // 16
            ts = lo2 + step * io16
            cnt = jnp.sum((eqidxs[b] <= ts).astype(jnp.int32),
                          axis=1, keepdims=True)
            num_lt = rsum((cnt < needs[b]).astype(jnp.int32))
            nlo.append(lo2 + num_lt * step)
            nhi.append(jnp.minimum(hi2, lo2 + (num_lt + 1) * step))
        return (tuple(nlo), tuple(nhi))

    z = jnp.zeros((1, 1), jnp.int32)
    los2, his2 = lax.fori_loop(
        0, 4, body_idx,
        (tuple(z - 1 for _ in range(B)),
         tuple(z + NPIX - 1 for _ in range(B))))

    for b in range(B):
        kth, jthr = los[b], his2[b]
        mask = (keys_list[b] > kth) | (eqs[b] & (idx <= jthr))
        out_ref[b, NC] = mask.astype(jnp.float32)


def _mlp_kernel(fine_ref, aux_ref, w1_ref, w2_ref, w3_ref, b1_ref, b2_ref,
                b3_ref, out_ref):
    fine = fine_ref[0].reshape(CF, BLK)
    aux = aux_ref[0].reshape(NC + 1, BLK)
    cu = aux[:NC]
    msk = aux[NC:NC + 1]
    pad = jnp.zeros((256 - CF - NC, BLK), jnp.bfloat16)
    x = jnp.concatenate([fine.astype(jnp.bfloat16), cu.astype(jnp.bfloat16),
                         pad], axis=0)
    h = jnp.dot(w1_ref[...].astype(jnp.bfloat16), x,
                preferred_element_type=jnp.float32)
    h = jnp.maximum(h + b1_ref[...], 0.0)
    h = jnp.dot(w2_ref[...].astype(jnp.bfloat16), h.astype(jnp.bfloat16),
                preferred_element_type=jnp.float32)
    h = jnp.maximum(h + b2_ref[...], 0.0)
    y = jnp.dot(w3_ref[...].astype(jnp.bfloat16), h.astype(jnp.bfloat16),
                preferred_element_type=jnp.float32)
    y = y + b3_ref[...]
    out_ref[0] = jnp.where(msk > 0.0, y, cu).reshape(NC, BLK // WF, WF)


def kernel(coarse_logits, fine_features, W1, b1, W2, b2, W3, b3):
    B = coarse_logits.shape[0]
    wy, wxt = _upsample_matrices()

    aux4 = pl.pallas_call(
        _select_kernel,
        grid=(1,),
        in_specs=[
            pl.BlockSpec((B, NC, HC, WC), lambda i: (0, 0, 0, 0)),
            pl.BlockSpec((HF, HC), lambda i: (0, 0)),
            pl.BlockSpec((HC, WF), lambda i: (0, 0)),
        ],
        out_specs=pl.BlockSpec((B, NC + 1, HF, WF), lambda i: (0, 0, 0, 0)),
        out_shape=jax.ShapeDtypeStruct((B, NC + 1, HF, WF), jnp.float32),
    )(coarse_logits, wy, wxt)

    w1p = jnp.pad(W1, ((0, 0), (0, 256 - W1.shape[1])))
    b1r = b1[:, None]
    b2r = b2[:, None]
    b3r = b3[:, None]

    out = pl.pallas_call(
        _mlp_kernel,
        grid=(B, NBLK),
        in_specs=[
            pl.BlockSpec((1, CF, BLK // WF, WF), lambda b, j: (b, 0, j, 0)),
            pl.BlockSpec((1, NC + 1, BLK // WF, WF), lambda b, j: (b, 0, j, 0)),
            pl.BlockSpec((256, 256), lambda b, j: (0, 0)),
            pl.BlockSpec((256, 256), lambda b, j: (0, 0)),
            pl.BlockSpec((NC, 256), lambda b, j: (0, 0)),
            pl.BlockSpec((256, 1), lambda b, j: (0, 0)),
            pl.BlockSpec((256, 1), lambda b, j: (0, 0)),
            pl.BlockSpec((NC, 1), lambda b, j: (0, 0)),
        ],
        out_specs=pl.BlockSpec((1, NC, BLK // WF, WF), lambda b, j: (b, 0, j, 0)),
        out_shape=jax.ShapeDtypeStruct((B, NC, HF, WF), jnp.float32),
    )(fine_features, aux4, w1p, W2, W3, b1r, b2r, b3r)

    return out


# speedup vs baseline: 1.2461x; 1.0168x over previous
import functools

import jax
import jax.numpy as jnp
from jax import lax
from jax.experimental import pallas as pl
from jax.experimental.pallas import tpu as pltpu

NUM_PTS = 2048
HF = 128
WF = 128
HC = 32
WC = 32
NC = 19
CF = 192
NPIX = HF * WF
BLK = 8192
ROWS = BLK // WF
NBLK = NPIX // BLK


def _upsample_matrices():
    def mat(out_n, in_n):
        s = jnp.linspace(0.0, in_n - 1.0, out_n)
        i0 = jnp.clip(jnp.floor(s), 0, in_n - 1)
        i1 = jnp.clip(i0 + 1, 0, in_n - 1)
        w = (s - i0)[:, None]
        oh0 = jax.nn.one_hot(i0.astype(jnp.int32), in_n, dtype=jnp.float32)
        oh1 = jax.nn.one_hot(i1.astype(jnp.int32), in_n, dtype=jnp.float32)
        return oh0 * (1.0 - w) + oh1 * w

    wy = mat(HF, HC)
    wxt = mat(WF, WC).T
    return wy, wxt


def _select_phase(coarse_ref, wy_ref, wxt_ref, aux_ref):
    wy = wy_ref[...]
    wxt = wxt_ref[...]
    kpts = jnp.int32(NUM_PTS)
    io16 = lax.broadcasted_iota(jnp.int32, (16, 1), 0) + 1
    iy = lax.broadcasted_iota(jnp.int32, (HF, WF), 0)
    ix = lax.broadcasted_iota(jnp.int32, (HF, WF), 1)
    idx = iy * WF + ix

    ups = []
    for ci in range(NC):
        a1 = jnp.dot(coarse_ref[0, ci], wxt,
                     precision=lax.Precision.HIGHEST,
                     preferred_element_type=jnp.float32)
        m = jnp.dot(wy, a1, precision=lax.Precision.HIGHEST,
                    preferred_element_type=jnp.float32)
        ups.append(m)
        aux_ref[ci] = m
    cmax = functools.reduce(jnp.maximum, ups)
    s = functools.reduce(jnp.add, [jnp.exp(u - cmax) for u in ups])
    unc = -(1.0 / s)
    keys = -lax.bitcast_convert_type(unc, jnp.int32)
    kflat = keys.reshape(1, NPIX)

    def rsum(x):
        return jnp.sum(x, axis=(0, 1), keepdims=True)

    lo0 = jnp.min(kflat, axis=(0, 1), keepdims=True)
    hi0 = jnp.max(kflat, axis=(0, 1), keepdims=True) + 1

    def body_val(_, carry):
        lo, hi = carry
        step = (hi - lo + 15) // 16
        ts = lo + step * io16
        cnt = jnp.sum((kflat >= ts).astype(jnp.int32),
                      axis=1, keepdims=True)
        num_ok = rsum((cnt >= kpts).astype(jnp.int32))
        return (lo + num_ok * step,
                jnp.minimum(hi, lo + (num_ok + 1) * step))

    kth, _ = lax.fori_loop(0, 7, body_val, (lo0, hi0))

    n_gt = rsum((kflat >= kth + 1).astype(jnp.int32))
    need = kpts - n_gt
    eq = keys == kth
    eqidx = jnp.where(eq, idx, jnp.int32(NPIX)).reshape(1, NPIX)

    def body_idx(_, carry):
        lo2, hi2 = carry
        step = (hi2 - lo2 + 15) // 16
        ts = lo2 + step * io16
        cnt = jnp.sum((eqidx <= ts).astype(jnp.int32),
                      axis=1, keepdims=True)
        num_lt = rsum((cnt < need).astype(jnp.int32))
        return (lo2 + num_lt * step,
                jnp.minimum(hi2, lo2 + (num_lt + 1) * step))

    z = jnp.zeros((1, 1), jnp.int32)
    _, jthr = lax.fori_loop(0, 4, body_idx, (z - 1, z + NPIX - 1))

    mask = (keys > kth) | (eq & (idx <= jthr))
    aux_ref[NC] = mask.astype(jnp.float32)


def _fused_kernel(coarse_ref, wy_ref, wxt_ref, fine_ref, w1_ref, w2_ref,
                  w3_ref, b1_ref, b2_ref, b3_ref, out_ref, aux_ref):
    j = pl.program_id(1)

    @pl.when(j == 0)
    def _():
        _select_phase(coarse_ref, wy_ref, wxt_ref, aux_ref)

    fine = fine_ref[0].reshape(CF, BLK)
    aux = aux_ref[:, pl.ds(j * ROWS, ROWS), :].reshape(NC + 1, BLK)
    cu = aux[:NC]
    msk = aux[NC:NC + 1]
    pad = jnp.zeros((256 - CF - NC, BLK), jnp.bfloat16)
    x = jnp.concatenate([fine.astype(jnp.bfloat16), cu.astype(jnp.bfloat16),
                         pad], axis=0)
    h = jnp.dot(w1_ref[...].astype(jnp.bfloat16), x,
                preferred_element_type=jnp.float32)
    h = jnp.maximum(h + b1_ref[...], 0.0)
    h = jnp.dot(w2_ref[...].astype(jnp.bfloat16), h.astype(jnp.bfloat16),
                preferred_element_type=jnp.float32)
    h = jnp.maximum(h + b2_ref[...], 0.0)
    y = jnp.dot(w3_ref[...].astype(jnp.bfloat16), h.astype(jnp.bfloat16),
                preferred_element_type=jnp.float32)
    y = y + b3_ref[...]
    out_ref[0] = jnp.where(msk > 0.0, y, cu).reshape(NC, ROWS, WF)


def kernel(coarse_logits, fine_features, W1, b1, W2, b2, W3, b3):
    B = coarse_logits.shape[0]
    wy, wxt = _upsample_matrices()
    w1p = jnp.pad(W1, ((0, 0), (0, 256 - W1.shape[1])))
    b1r = b1[:, None]
    b2r = b2[:, None]
    b3r = b3[:, None]

    out = pl.pallas_call(
        _fused_kernel,
        grid=(B, NBLK),
        in_specs=[
            pl.BlockSpec((1, NC, HC, WC), lambda b, j: (b, 0, 0, 0)),
            pl.BlockSpec((HF, HC), lambda b, j: (0, 0)),
            pl.BlockSpec((HC, WF), lambda b, j: (0, 0)),
            pl.BlockSpec((1, CF, ROWS, WF), lambda b, j: (b, 0, j, 0)),
            pl.BlockSpec((256, 256), lambda b, j: (0, 0)),
            pl.BlockSpec((256, 256), lambda b, j: (0, 0)),
            pl.BlockSpec((NC, 256), lambda b, j: (0, 0)),
            pl.BlockSpec((256, 1), lambda b, j: (0, 0)),
            pl.BlockSpec((256, 1), lambda b, j: (0, 0)),
            pl.BlockSpec((NC, 1), lambda b, j: (0, 0)),
        ],
        out_specs=pl.BlockSpec((1, NC, ROWS, WF), lambda b, j: (b, 0, j, 0)),
        out_shape=jax.ShapeDtypeStruct((B, NC, HF, WF), jnp.float32),
        scratch_shapes=[pltpu.VMEM((NC + 1, HF, WF), jnp.float32)],
    )(coarse_logits, wy, wxt, fine_features, w1p, W2, W3, b1r, b2r, b3r)

    return out


# speedup vs baseline: 1.2544x; 1.0067x over previous
import functools

import jax
import jax.numpy as jnp
from jax import lax
from jax.experimental import pallas as pl
from jax.experimental.pallas import tpu as pltpu

NUM_PTS = 2048
HF = 128
WF = 128
HC = 32
WC = 32
NC = 19
CF = 192
NPIX = HF * WF
BLK = 2048
ROWS = BLK // WF
NBLK = NPIX // BLK


def _upsample_matrices():
    def mat(out_n, in_n):
        s = jnp.linspace(0.0, in_n - 1.0, out_n)
        i0 = jnp.clip(jnp.floor(s), 0, in_n - 1)
        i1 = jnp.clip(i0 + 1, 0, in_n - 1)
        w = (s - i0)[:, None]
        oh0 = jax.nn.one_hot(i0.astype(jnp.int32), in_n, dtype=jnp.float32)
        oh1 = jax.nn.one_hot(i1.astype(jnp.int32), in_n, dtype=jnp.float32)
        return oh0 * (1.0 - w) + oh1 * w

    wy = mat(HF, HC)
    wxt = mat(WF, WC).T
    return wy, wxt


def _select_phase(coarse_ref, wy_ref, wxt_ref, aux_ref):
    wy = wy_ref[...]
    wxt = wxt_ref[...]
    kpts = jnp.int32(NUM_PTS)
    io16 = lax.broadcasted_iota(jnp.int32, (16, 1), 0) + 1
    iy = lax.broadcasted_iota(jnp.int32, (HF, WF), 0)
    ix = lax.broadcasted_iota(jnp.int32, (HF, WF), 1)
    idx = iy * WF + ix

    ups = []
    for ci in range(NC):
        a1 = jnp.dot(coarse_ref[0, ci], wxt,
                     precision=lax.Precision.HIGHEST,
                     preferred_element_type=jnp.float32)
        m = jnp.dot(wy, a1, precision=lax.Precision.HIGHEST,
                    preferred_element_type=jnp.float32)
        ups.append(m)
        aux_ref[ci] = m
    cmax = functools.reduce(jnp.maximum, ups)
    s = functools.reduce(jnp.add, [jnp.exp(u - cmax) for u in ups])
    unc = -(1.0 / s)
    keys = -lax.bitcast_convert_type(unc, jnp.int32)
    kflat = keys.reshape(1, NPIX)

    def rsum(x):
        return jnp.sum(x, axis=(0, 1), keepdims=True)

    lo0 = jnp.min(kflat, axis=(0, 1), keepdims=True)
    hi0 = jnp.max(kflat, axis=(0, 1), keepdims=True) + 1

    def body_val(_, carry):
        lo, hi = carry
        step = (hi - lo + 15) // 16
        ts = lo + step * io16
        cnt = jnp.sum((kflat >= ts).astype(jnp.int32),
                      axis=1, keepdims=True)
        num_ok = rsum((cnt >= kpts).astype(jnp.int32))
        return (lo + num_ok * step,
                jnp.minimum(hi, lo + (num_ok + 1) * step))

    kth, _ = lax.fori_loop(0, 7, body_val, (lo0, hi0))

    n_gt = rsum((kflat >= kth + 1).astype(jnp.int32))
    need = kpts - n_gt
    eq = keys == kth
    eqidx = jnp.where(eq, idx, jnp.int32(NPIX)).reshape(1, NPIX)

    def body_idx(_, carry):
        lo2, hi2 = carry
        step = (hi2 - lo2 + 15) // 16
        ts = lo2 + step * io16
        cnt = jnp.sum((eqidx <= ts).astype(jnp.int32),
                      axis=1, keepdims=True)
        num_lt = rsum((cnt < need).astype(jnp.int32))
        return (lo2 + num_lt * step,
                jnp.minimum(hi2, lo2 + (num_lt + 1) * step))

    z = jnp.zeros((1, 1), jnp.int32)
    _, jthr = lax.fori_loop(0, 4, body_idx, (z - 1, z + NPIX - 1))

    mask = (keys > kth) | (eq & (idx <= jthr))
    aux_ref[NC] = mask.astype(jnp.float32)


def _fused_kernel(coarse_ref, wy_ref, wxt_ref, fine_ref, w1_ref, w2_ref,
                  w3_ref, b1_ref, b2_ref, b3_ref, out_ref, aux_ref):
    j = pl.program_id(1)

    @pl.when(j == 0)
    def _():
        _select_phase(coarse_ref, wy_ref, wxt_ref, aux_ref)

    fine = fine_ref[0].reshape(CF, BLK)
    aux = aux_ref[:, pl.ds(j * ROWS, ROWS), :].reshape(NC + 1, BLK)
    cu = aux[:NC]
    msk = aux[NC:NC + 1]
    pad = jnp.zeros((256 - CF - NC, BLK), jnp.bfloat16)
    x = jnp.concatenate([fine.astype(jnp.bfloat16), cu.astype(jnp.bfloat16),
                         pad], axis=0)
    h = jnp.dot(w1_ref[...].astype(jnp.bfloat16), x,
                preferred_element_type=jnp.float32)
    h = jnp.maximum(h + b1_ref[...], 0.0)
    h = jnp.dot(w2_ref[...].astype(jnp.bfloat16), h.astype(jnp.bfloat16),
                preferred_element_type=jnp.float32)
    h = jnp.maximum(h + b2_ref[...], 0.0)
    y = jnp.dot(w3_ref[...].astype(jnp.bfloat16), h.astype(jnp.bfloat16),
                preferred_element_type=jnp.float32)
    y = y + b3_ref[...]
    out_ref[0] = jnp.where(msk > 0.0, y, cu).reshape(NC, ROWS, WF)


def kernel(coarse_logits, fine_features, W1, b1, W2, b2, W3, b3):
    B = coarse_logits.shape[0]
    wy, wxt = _upsample_matrices()
    w1p = jnp.pad(W1, ((0, 0), (0, 256 - W1.shape[1])))
    b1r = b1[:, None]
    b2r = b2[:, None]
    b3r = b3[:, None]

    out = pl.pallas_call(
        _fused_kernel,
        grid=(B, NBLK),
        in_specs=[
            pl.BlockSpec((1, NC, HC, WC), lambda b, j: (b, 0, 0, 0)),
            pl.BlockSpec((HF, HC), lambda b, j: (0, 0)),
            pl.BlockSpec((HC, WF), lambda b, j: (0, 0)),
            pl.BlockSpec((1, CF, ROWS, WF), lambda b, j: (b, 0, j, 0)),
            pl.BlockSpec((256, 256), lambda b, j: (0, 0)),
            pl.BlockSpec((256, 256), lambda b, j: (0, 0)),
            pl.BlockSpec((NC, 256), lambda b, j: (0, 0)),
            pl.BlockSpec((256, 1), lambda b, j: (0, 0)),
            pl.BlockSpec((256, 1), lambda b, j: (0, 0)),
            pl.BlockSpec((NC, 1), lambda b, j: (0, 0)),
        ],
        out_specs=pl.BlockSpec((1, NC, ROWS, WF), lambda b, j: (b, 0, j, 0)),
        out_shape=jax.ShapeDtypeStruct((B, NC, HF, WF), jnp.float32),
        scratch_shapes=[pltpu.VMEM((NC + 1, HF, WF), jnp.float32)],
    )(coarse_logits, wy, wxt, fine_features, w1p, W2, W3, b1r, b2r, b3r)

    return out


# speedup vs baseline: 1.3446x; 1.0719x over previous
import functools

import jax
import jax.numpy as jnp
from jax import lax
from jax.experimental import pallas as pl
from jax.experimental.pallas import tpu as pltpu

NUM_PTS = 2048
HF = 128
WF = 128
HC = 32
WC = 32
NC = 19
CF = 192
NPIX = HF * WF
BLK = 2048
ROWS = BLK // WF
NBLK = NPIX // BLK


def _upsample_matrices():
    def mat(out_n, in_n):
        s = jnp.linspace(0.0, in_n - 1.0, out_n)
        i0 = jnp.clip(jnp.floor(s), 0, in_n - 1)
        i1 = jnp.clip(i0 + 1, 0, in_n - 1)
        w = (s - i0)[:, None]
        oh0 = jax.nn.one_hot(i0.astype(jnp.int32), in_n, dtype=jnp.float32)
        oh1 = jax.nn.one_hot(i1.astype(jnp.int32), in_n, dtype=jnp.float32)
        return oh0 * (1.0 - w) + oh1 * w

    wy = mat(HF, HC)
    wxt = mat(WF, WC).T
    return wy, wxt


def _select_phase(coarse_ref, wy_ref, wxt_ref, aux_ref):
    wy = wy_ref[...]
    wxt = wxt_ref[...]
    kpts = jnp.int32(NUM_PTS)
    io16 = lax.broadcasted_iota(jnp.int32, (16, 1), 0) + 1
    iy = lax.broadcasted_iota(jnp.int32, (HF, WF), 0)
    ix = lax.broadcasted_iota(jnp.int32, (HF, WF), 1)
    idx = iy * WF + ix

    ups = []
    for ci in range(NC):
        a1 = jnp.dot(coarse_ref[0, ci], wxt,
                     precision=lax.Precision.HIGHEST,
                     preferred_element_type=jnp.float32)
        m = jnp.dot(wy, a1, precision=lax.Precision.HIGHEST,
                    preferred_element_type=jnp.float32)
        ups.append(m)
        aux_ref[ci] = m
    cmax = functools.reduce(jnp.maximum, ups)
    s = functools.reduce(jnp.add, [jnp.exp(u - cmax) for u in ups])
    unc = -(1.0 / s)
    keys = -lax.bitcast_convert_type(unc, jnp.int32)
    kflat = keys.reshape(1, NPIX)

    def rsum(x):
        return jnp.sum(x, axis=(0, 1), keepdims=True)

    lo0 = jnp.min(kflat, axis=(0, 1), keepdims=True)
    hi0 = jnp.max(kflat, axis=(0, 1), keepdims=True) + 1

    def body_val(_, carry):
        lo, hi = carry
        step = (hi - lo + 15) // 16
        ts = lo + step * io16
        cnt = jnp.sum((kflat >= ts).astype(jnp.int32),
                      axis=1, keepdims=True)
        num_ok = rsum((cnt >= kpts).astype(jnp.int32))
        return (lo + num_ok * step,
                jnp.minimum(hi, lo + (num_ok + 1) * step))

    kth, _ = lax.fori_loop(0, 7, body_val, (lo0, hi0))

    n_gt = rsum((kflat >= kth + 1).astype(jnp.int32))
    need = kpts - n_gt
    eq = keys == kth
    eqidx = jnp.where(eq, idx, jnp.int32(NPIX)).reshape(1, NPIX)

    def body_idx(_, carry):
        lo2, hi2 = carry
        step = (hi2 - lo2 + 15) // 16
        ts = lo2 + step * io16
        cnt = jnp.sum((eqidx <= ts).astype(jnp.int32),
                      axis=1, keepdims=True)
        num_lt = rsum((cnt < need).astype(jnp.int32))
        return (lo2 + num_lt * step,
                jnp.minimum(hi2, lo2 + (num_lt + 1) * step))

    z = jnp.zeros((1, 1), jnp.int32)
    _, jthr = lax.fori_loop(0, 4, body_idx, (z - 1, z + NPIX - 1))

    mask = (keys > kth) | (eq & (idx <= jthr))
    aux_ref[NC] = mask.astype(jnp.float32)


NBUF = 4


def _fused_kernel(coarse_ref, wy_ref, wxt_ref, fine_ref, w1_ref, w2_ref,
                  w3_ref, b1_ref, b2_ref, b3_ref, out_ref, aux_ref,
                  fbuf_ref, sem_ref):
    b = pl.program_id(0)

    def dma(k, slot):
        return pltpu.make_async_copy(
            fine_ref.at[b, :, pl.ds(k * ROWS, ROWS), :],
            fbuf_ref.at[slot],
            sem_ref.at[slot])

    for k in range(NBUF):
        dma(k, k).start()

    _select_phase(coarse_ref, wy_ref, wxt_ref, aux_ref)

    for k in range(NBLK):
        slot = k % NBUF
        dma(k, slot).wait()
        fine = fbuf_ref[slot].reshape(CF, BLK)
        aux = aux_ref[:, k * ROWS:(k + 1) * ROWS, :].reshape(NC + 1, BLK)
        cu = aux[:NC]
        msk = aux[NC:NC + 1]
        pad = jnp.zeros((256 - CF - NC, BLK), jnp.bfloat16)
        x = jnp.concatenate([fine.astype(jnp.bfloat16),
                             cu.astype(jnp.bfloat16), pad], axis=0)
        h = jnp.dot(w1_ref[...].astype(jnp.bfloat16), x,
                    preferred_element_type=jnp.float32)
        h = jnp.maximum(h + b1_ref[...], 0.0)
        h = jnp.dot(w2_ref[...].astype(jnp.bfloat16), h.astype(jnp.bfloat16),
                    preferred_element_type=jnp.float32)
        h = jnp.maximum(h + b2_ref[...], 0.0)
        y = jnp.dot(w3_ref[...].astype(jnp.bfloat16), h.astype(jnp.bfloat16),
                    preferred_element_type=jnp.float32)
        y = y + b3_ref[...]
        if k + NBUF < NBLK:
            dma(k + NBUF, slot).start()
        out_ref[0, :, k * ROWS:(k + 1) * ROWS, :] = (
            jnp.where(msk > 0.0, y, cu).reshape(NC, ROWS, WF))


def kernel(coarse_logits, fine_features, W1, b1, W2, b2, W3, b3):
    B = coarse_logits.shape[0]
    wy, wxt = _upsample_matrices()
    w1p = jnp.pad(W1, ((0, 0), (0, 256 - W1.shape[1])))
    b1r = b1[:, None]
    b2r = b2[:, None]
    b3r = b3[:, None]

    out = pl.pallas_call(
        _fused_kernel,
        grid=(B,),
        in_specs=[
            pl.BlockSpec((1, NC, HC, WC), lambda b: (b, 0, 0, 0)),
            pl.BlockSpec((HF, HC), lambda b: (0, 0)),
            pl.BlockSpec((HC, WF), lambda b: (0, 0)),
            pl.BlockSpec(memory_space=pl.ANY),
            pl.BlockSpec((256, 256), lambda b: (0, 0)),
            pl.BlockSpec((256, 256), lambda b: (0, 0)),
            pl.BlockSpec((NC, 256), lambda b: (0, 0)),
            pl.BlockSpec((256, 1), lambda b: (0, 0)),
            pl.BlockSpec((256, 1), lambda b: (0, 0)),
            pl.BlockSpec((NC, 1), lambda b: (0, 0)),
        ],
        out_specs=pl.BlockSpec((1, NC, HF, WF), lambda b: (b, 0, 0, 0)),
        out_shape=jax.ShapeDtypeStruct((B, NC, HF, WF), jnp.float32),
        scratch_shapes=[
            pltpu.VMEM((NC + 1, HF, WF), jnp.float32),
            pltpu.VMEM((NBUF, CF, ROWS, WF), jnp.float32),
            pltpu.SemaphoreType.DMA((NBUF,)),
        ],
    )(coarse_logits, wy, wxt, fine_features, w1p, W2, W3, b1r, b2r, b3r)

    return out
